# baseline (speedup 1.0000x reference)
import functools

import jax
import jax.numpy as jnp
from jax import lax
from jax.experimental import pallas as pl
from jax.experimental.pallas import tpu as pltpu
from jax.experimental.pallas import tpu_sc as plsc

B, N, K, M, D = 2, 10000, 32, 10000, 256
NC, NS, LANES = 2, 16, 16
NW = NC * NS
E = B * N * K
EPW = E // NW
CH = 10000
NCHUNK = EPW // CH


def _sc_gather(table_flat, idx_flat):
    mesh = plsc.VectorSubcoreMesh(core_axis_name="c", subcore_axis_name="s")

    @functools.partial(
        pl.kernel,
        out_type=(jax.ShapeDtypeStruct((E,), jnp.float32),) * 3,
        mesh=mesh,
        compiler_params=pltpu.CompilerParams(needs_layout_passes=False),
        scratch_types=[
            pltpu.VMEM((3 * M,), jnp.float32),
            pltpu.VMEM((CH,), jnp.int32),
            pltpu.VMEM((CH,), jnp.float32),
            pltpu.VMEM((CH,), jnp.float32),
            pltpu.VMEM((CH,), jnp.float32),
        ],
    )
    def run(tab_hbm, idx_hbm, qx_hbm, qy_hbm, qz_hbm, tab_v, idx_v, qxv, qyv, qzv):
        wid = lax.axis_index("s") * NC + lax.axis_index("c")
        pltpu.sync_copy(tab_hbm, tab_v)
        base0 = wid * EPW
        for c in range(NCHUNK):
            base = base0 + c * CH
            pltpu.sync_copy(idx_hbm.at[pl.ds(base, CH)], idx_v)

            @plsc.parallel_loop(0, CH, LANES, unroll=8)
            def body(off):
                b3 = idx_v[pl.ds(off, LANES)] * 3
                qxv[pl.ds(off, LANES)] = plsc.load_gather(tab_v, [b3])
                qyv[pl.ds(off, LANES)] = plsc.load_gather(tab_v, [b3 + 1])
                qzv[pl.ds(off, LANES)] = plsc.load_gather(tab_v, [b3 + 2])
            pltpu.sync_copy(qxv, qx_hbm.at[pl.ds(base, CH)])
            pltpu.sync_copy(qyv, qy_hbm.at[pl.ds(base, CH)])
            pltpu.sync_copy(qzv, qz_hbm.at[pl.ds(base, CH)])

    return run(table_flat, idx_flat)


BN = 2000


def _tc_one(p, qx, qy, qz, w, wbig):
    px = p[:, 0:1]
    py = p[:, 1:2]
    pz = p[:, 2:3]
    dx = px - qx
    dy = py - qy
    dz = pz - qz
    dist = jnp.sqrt(dx * dx + dy * dy + dz * dz)
    h = jnp.bfloat16
    xb = jnp.concatenate(
        [qx.astype(h), qy.astype(h), qz.astype(h), dist.astype(h)],
        axis=1)
    y = jax.lax.dot_general(
        xb, wbig, (((1,), (0,)), ((), ())),
        preferred_element_type=jnp.float32)
    acc = y[:, 0:D]
    for k in range(1, K):
        acc = jnp.maximum(acc, y[:, k * D:(k + 1) * D])
    return (acc
            + px * w[0:1, :] + py * w[1:2, :] + pz * w[2:3, :] + w[3:4, :])


def _tc_body(p_ref, qx_ref, qy_ref, qz_ref, w_ref, wbig_ref, o0_ref, o1_ref):
    w = w_ref[...]
    wbig = wbig_ref[...]
    o0_ref[...] = _tc_one(p_ref[0], qx_ref[0], qy_ref[0], qz_ref[0], w, wbig)
    o1_ref[...] = _tc_one(p_ref[1], qx_ref[1], qy_ref[1], qz_ref[1], w, wbig)


def _tc_call(p, qx, qy, qz, wpack, wbig):
    return pl.pallas_call(
        _tc_body,
        grid=(N // BN,),
        in_specs=[
            pl.BlockSpec((B, BN, 3), lambda i: (0, i, 0)),
            pl.BlockSpec((B, BN, K), lambda i: (0, i, 0)),
            pl.BlockSpec((B, BN, K), lambda i: (0, i, 0)),
            pl.BlockSpec((B, BN, K), lambda i: (0, i, 0)),
            pl.BlockSpec((4, D), lambda i: (0, 0)),
            pl.BlockSpec((4 * K, K * D), lambda i: (0, 0)),
        ],
        out_specs=[
            pl.BlockSpec((BN, D), lambda i: (i, 0)),
            pl.BlockSpec((BN, D), lambda i: (i, 0)),
        ],
        out_shape=[
            jax.ShapeDtypeStruct((N, D), jnp.float32),
            jax.ShapeDtypeStruct((N, D), jnp.float32),
        ],
    )(p, qx, qy, qz, wpack, wbig)


def kernel(points_xyz, index, all_points_xyz, W, b):
    tab = all_points_xyz.reshape(-1)
    idx = index.reshape(-1)
    qx, qy, qz = _sc_gather(tab, idx)
    qx = qx.reshape(B, N, K)
    qy = qy.reshape(B, N, K)
    qz = qz.reshape(B, N, K)
    A = W[:, 0:3] + W[:, 6:9]
    Bq = W[:, 3:6] - W[:, 6:9]
    wpack = jnp.concatenate([A.T, b[None, :]], axis=0)
    w4 = jnp.concatenate([Bq.T, W[:, 9:10].T], axis=0)
    w4h = w4.astype(jnp.bfloat16)
    eyeK = jnp.eye(K, dtype=jnp.bfloat16)
    wbig = (eyeK[None, :, :, None] * w4h[:, None, None, :]).reshape(
        4 * K, K * D)
    out0, out1 = _tc_call(points_xyz, qx, qy, qz, wpack, wbig)
    return (out0, out1)


# speedup vs baseline: 1.0036x; 1.0036x over previous
import functools

import jax
import jax.numpy as jnp
from jax import lax
from jax.experimental import pallas as pl
from jax.experimental.pallas import tpu as pltpu
from jax.experimental.pallas import tpu_sc as plsc

B, N, K, M, D = 2, 10000, 32, 10000, 256
NC, NS, LANES = 2, 16, 16
NW = NC * NS
E = B * N * K
EPW = E // NW
CH = 20000
NCHUNK = EPW // CH


def _sc_gather(table_flat, idx_flat):
    mesh = plsc.VectorSubcoreMesh(core_axis_name="c", subcore_axis_name="s")

    @functools.partial(
        pl.kernel,
        out_type=(jax.ShapeDtypeStruct((E,), jnp.float32),) * 3,
        mesh=mesh,
        compiler_params=pltpu.CompilerParams(needs_layout_passes=False),
        scratch_types=[
            pltpu.VMEM((3 * M,), jnp.float32),
            pltpu.VMEM((CH,), jnp.int32),
            pltpu.VMEM((CH,), jnp.float32),
            pltpu.VMEM((CH,), jnp.float32),
            pltpu.VMEM((CH,), jnp.float32),
        ],
    )
    def run(tab_hbm, idx_hbm, qx_hbm, qy_hbm, qz_hbm, tab_v, idx_v, qxv, qyv, qzv):
        wid = lax.axis_index("s") * NC + lax.axis_index("c")
        pltpu.sync_copy(tab_hbm, tab_v)
        base0 = wid * EPW
        for c in range(NCHUNK):
            base = base0 + c * CH
            pltpu.sync_copy(idx_hbm.at[pl.ds(base, CH)], idx_v)

            @plsc.parallel_loop(0, CH, LANES, unroll=8)
            def body(off):
                b3 = idx_v[pl.ds(off, LANES)] * 3
                qxv[pl.ds(off, LANES)] = plsc.load_gather(tab_v, [b3])
                qyv[pl.ds(off, LANES)] = plsc.load_gather(tab_v, [b3 + 1])
                qzv[pl.ds(off, LANES)] = plsc.load_gather(tab_v, [b3 + 2])
            pltpu.sync_copy(qxv, qx_hbm.at[pl.ds(base, CH)])
            pltpu.sync_copy(qyv, qy_hbm.at[pl.ds(base, CH)])
            pltpu.sync_copy(qzv, qz_hbm.at[pl.ds(base, CH)])

    return run(table_flat, idx_flat)


BN = 1000


def _tc_one(p, qx, qy, qz, w, wbig):
    px = p[:, 0:1]
    py = p[:, 1:2]
    pz = p[:, 2:3]
    dx = px - qx
    dy = py - qy
    dz = pz - qz
    dist = jnp.sqrt(dx * dx + dy * dy + dz * dz)
    h = jnp.bfloat16
    xb = jnp.concatenate(
        [qx.astype(h), qy.astype(h), qz.astype(h), dist.astype(h)],
        axis=1)
    y = jax.lax.dot_general(
        xb, wbig, (((1,), (0,)), ((), ())),
        preferred_element_type=jnp.float32)
    acc = y[:, 0:D]
    for k in range(1, K):
        acc = jnp.maximum(acc, y[:, k * D:(k + 1) * D])
    return (acc
            + px * w[0:1, :] + py * w[1:2, :] + pz * w[2:3, :] + w[3:4, :])


def _tc_body(p_ref, qx_ref, qy_ref, qz_ref, w_ref, wbig_ref, o0_ref, o1_ref):
    w = w_ref[...]
    wbig = wbig_ref[...]
    o0_ref[...] = _tc_one(p_ref[0], qx_ref[0], qy_ref[0], qz_ref[0], w, wbig)
    o1_ref[...] = _tc_one(p_ref[1], qx_ref[1], qy_ref[1], qz_ref[1], w, wbig)


def _tc_call(p, qx, qy, qz, wpack, wbig):
    return pl.pallas_call(
        _tc_body,
        grid=(N // BN,),
        in_specs=[
            pl.BlockSpec((B, BN, 3), lambda i: (0, i, 0)),
            pl.BlockSpec((B, BN, K), lambda i: (0, i, 0)),
            pl.BlockSpec((B, BN, K), lambda i: (0, i, 0)),
            pl.BlockSpec((B, BN, K), lambda i: (0, i, 0)),
            pl.BlockSpec((4, D), lambda i: (0, 0)),
            pl.BlockSpec((4 * K, K * D), lambda i: (0, 0)),
        ],
        out_specs=[
            pl.BlockSpec((BN, D), lambda i: (i, 0)),
            pl.BlockSpec((BN, D), lambda i: (i, 0)),
        ],
        out_shape=[
            jax.ShapeDtypeStruct((N, D), jnp.float32),
            jax.ShapeDtypeStruct((N, D), jnp.float32),
        ],
    )(p, qx, qy, qz, wpack, wbig)


def kernel(points_xyz, index, all_points_xyz, W, b):
    tab = all_points_xyz.reshape(-1)
    idx = index.reshape(-1)
    qx, qy, qz = _sc_gather(tab, idx)
    qx = qx.reshape(B, N, K)
    qy = qy.reshape(B, N, K)
    qz = qz.reshape(B, N, K)
    A = W[:, 0:3] + W[:, 6:9]
    Bq = W[:, 3:6] - W[:, 6:9]
    wpack = jnp.concatenate([A.T, b[None, :]], axis=0)
    w4 = jnp.concatenate([Bq.T, W[:, 9:10].T], axis=0)
    w4h = w4.astype(jnp.bfloat16)
    eyeK = jnp.eye(K, dtype=jnp.bfloat16)
    wbig = (eyeK[None, :, :, None] * w4h[:, None, None, :]).reshape(
        4 * K, K * D)
    out0, out1 = _tc_call(points_xyz, qx, qy, qz, wpack, wbig)
    return (out0, out1)


# speedup vs baseline: 1.0054x; 1.0018x over previous
import functools

import jax
import jax.numpy as jnp
from jax import lax
from jax.experimental import pallas as pl
from jax.experimental.pallas import tpu as pltpu
from jax.experimental.pallas import tpu_sc as plsc

B, N, K, M, D = 2, 10000, 32, 10000, 256
NC, NS, LANES = 2, 16, 16
NW = NC * NS
E = B * N * K
EPW = E // NW
CH = 20000
NCHUNK = EPW // CH


def _sc_gather(table_flat, idx_flat):
    mesh = plsc.VectorSubcoreMesh(core_axis_name="c", subcore_axis_name="s")

    @functools.partial(
        pl.kernel,
        out_type=(jax.ShapeDtypeStruct((E,), jnp.float32),) * 3,
        mesh=mesh,
        compiler_params=pltpu.CompilerParams(needs_layout_passes=False),
        scratch_types=[
            pltpu.VMEM((3 * M,), jnp.float32),
            pltpu.VMEM((CH,), jnp.int32),
            pltpu.VMEM((CH,), jnp.float32),
            pltpu.VMEM((CH,), jnp.float32),
            pltpu.VMEM((CH,), jnp.float32),
        ],
    )
    def run(tab_hbm, idx_hbm, qx_hbm, qy_hbm, qz_hbm, tab_v, idx_v, qxv, qyv, qzv):
        wid = lax.axis_index("s") * NC + lax.axis_index("c")
        pltpu.sync_copy(tab_hbm, tab_v)
        base0 = wid * EPW
        for c in range(NCHUNK):
            base = base0 + c * CH
            pltpu.sync_copy(idx_hbm.at[pl.ds(base, CH)], idx_v)

            @plsc.parallel_loop(0, CH, LANES, unroll=16)
            def body(off):
                b3 = idx_v[pl.ds(off, LANES)] * 3
                qxv[pl.ds(off, LANES)] = plsc.load_gather(tab_v, [b3])
                qyv[pl.ds(off, LANES)] = plsc.load_gather(tab_v, [b3 + 1])
                qzv[pl.ds(off, LANES)] = plsc.load_gather(tab_v, [b3 + 2])
            pltpu.sync_copy(qxv, qx_hbm.at[pl.ds(base, CH)])
            pltpu.sync_copy(qyv, qy_hbm.at[pl.ds(base, CH)])
            pltpu.sync_copy(qzv, qz_hbm.at[pl.ds(base, CH)])

    return run(table_flat, idx_flat)


BN = 1000


def _tc_one(p, qx, qy, qz, w, wbig):
    px = p[:, 0:1]
    py = p[:, 1:2]
    pz = p[:, 2:3]
    dx = px - qx
    dy = py - qy
    dz = pz - qz
    dist = jnp.sqrt(dx * dx + dy * dy + dz * dz)
    h = jnp.bfloat16
    xb = jnp.concatenate(
        [qx.astype(h), qy.astype(h), qz.astype(h), dist.astype(h)],
        axis=1)
    y = jax.lax.dot_general(
        xb, wbig, (((1,), (0,)), ((), ())),
        preferred_element_type=jnp.float32)
    acc = y[:, 0:D]
    for k in range(1, K):
        acc = jnp.maximum(acc, y[:, k * D:(k + 1) * D])
    return (acc
            + px * w[0:1, :] + py * w[1:2, :] + pz * w[2:3, :] + w[3:4, :])


def _tc_body(p_ref, qx_ref, qy_ref, qz_ref, w_ref, wbig_ref, o0_ref, o1_ref):
    w = w_ref[...]
    wbig = wbig_ref[...]
    o0_ref[...] = _tc_one(p_ref[0], qx_ref[0], qy_ref[0], qz_ref[0], w, wbig)
    o1_ref[...] = _tc_one(p_ref[1], qx_ref[1], qy_ref[1], qz_ref[1], w, wbig)


def _tc_call(p, qx, qy, qz, wpack, wbig):
    return pl.pallas_call(
        _tc_body,
        grid=(N // BN,),
        in_specs=[
            pl.BlockSpec((B, BN, 3), lambda i: (0, i, 0)),
            pl.BlockSpec((B, BN, K), lambda i: (0, i, 0)),
            pl.BlockSpec((B, BN, K), lambda i: (0, i, 0)),
            pl.BlockSpec((B, BN, K), lambda i: (0, i, 0)),
            pl.BlockSpec((4, D), lambda i: (0, 0)),
            pl.BlockSpec((4 * K, K * D), lambda i: (0, 0)),
        ],
        out_specs=[
            pl.BlockSpec((BN, D), lambda i: (i, 0)),
            pl.BlockSpec((BN, D), lambda i: (i, 0)),
        ],
        out_shape=[
            jax.ShapeDtypeStruct((N, D), jnp.float32),
            jax.ShapeDtypeStruct((N, D), jnp.float32),
        ],
    )(p, qx, qy, qz, wpack, wbig)


def kernel(points_xyz, index, all_points_xyz, W, b):
    tab = all_points_xyz.reshape(-1)
    idx = index.reshape(-1)
    qx, qy, qz = _sc_gather(tab, idx)
    qx = qx.reshape(B, N, K)
    qy = qy.reshape(B, N, K)
    qz = qz.reshape(B, N, K)
    A = W[:, 0:3] + W[:, 6:9]
    Bq = W[:, 3:6] - W[:, 6:9]
    wpack = jnp.concatenate([A.T, b[None, :]], axis=0)
    w4 = jnp.concatenate([Bq.T, W[:, 9:10].T], axis=0)
    w4h = w4.astype(jnp.bfloat16)
    eyeK = jnp.eye(K, dtype=jnp.bfloat16)
    wbig = (eyeK[None, :, :, None] * w4h[:, None, None, :]).reshape(
        4 * K, K * D)
    out0, out1 = _tc_call(points_xyz, qx, qy, qz, wpack, wbig)
    return (out0, out1)
